# baseline (speedup 1.0000x reference)
import jax
import jax.numpy as jnp
from jax.experimental import pallas as pl

N_EP = 10000
N_FL = 10000
E = 320000
F_IN = 128
H = 4
D = 32


def _gat(feat_src, feat_dst, src, dst, W, attn_l, attn_r, bias, num_dst):
    hs = (feat_src @ W).reshape(-1, H, D)
    hd = (feat_dst @ W).reshape(-1, H, D)
    el = (hs * attn_l[None, :, :]).sum(-1)
    er = (hd * attn_r[None, :, :]).sum(-1)
    e = jax.nn.leaky_relu(el[src] + er[dst], negative_slope=0.2)
    m = jax.ops.segment_max(e, dst, num_segments=num_dst)
    m = jnp.where(jnp.isfinite(m), m, 0.0)
    ex = jnp.exp(e - m[dst])
    denom = jax.ops.segment_sum(ex, dst, num_segments=num_dst)
    alpha = ex / denom[dst]
    out = jax.ops.segment_sum(alpha[:, :, None] * hs[src], dst, num_segments=num_dst)
    out = out + bias.reshape(1, H, D)
    return out.reshape(num_dst, H * D)


def kernel(x_endpoint, x_flow, edge_index_ep2flow, edge_index_flow2ep,
           W1, attn_l1, attn_r1, b1, W2, attn_l2, attn_r2, b2):
    h_flow = _gat(x_endpoint, x_flow, edge_index_ep2flow[0], edge_index_ep2flow[1],
                  W1, attn_l1, attn_r1, b1, N_FL)
    h_endpoint = _gat(x_flow, x_endpoint, edge_index_flow2ep[0], edge_index_flow2ep[1],
                      W2, attn_l2, attn_r2, b2, N_EP)
    h_flow = h_flow + x_flow
    h_endpoint = h_endpoint + x_endpoint
    return (h_flow, h_endpoint)


# speedup vs baseline: 66.1454x; 66.1454x over previous
import dataclasses
import functools

import jax
import jax.numpy as jnp
from jax import lax
from jax.experimental import pallas as pl
from jax.experimental.pallas import tpu as pltpu
from jax.experimental.pallas import tpu_sc as plsc

N = 10000
E = 320000
F = 128
H = 4
D = 32
TW = 48
SLOPE = 0.2

NTILE = 16
NCORE = 2
EPT = E // NTILE
CHUNK = 400
NCHUNK = EPT // CHUNK
SLEN = 80
NSTR = CHUNK // SLEN
ACC_ROWS = 2 * N
ZTILES = 10
COPY_ROWS = ACC_ROWS // ZTILES


NB = 1000
NBLK = N // NB


def _proj_body(fs_ref, fd_ref, w_ref, al_ref, ar_ref,
               hs_ref, el_ref, er_ref):
    w = w_ref[...]
    hs = jnp.dot(fs_ref[...], w, preferred_element_type=jnp.float32)
    hd = jnp.dot(fd_ref[...], w, preferred_element_type=jnp.float32)
    hs_ref[...] = hs
    al = al_ref[...]
    ar = ar_ref[...]
    el_ref[...] = jnp.concatenate(
        [jnp.sum(hs[:, h * D:(h + 1) * D] * al[h:h + 1, :], axis=1,
                 keepdims=True) for h in range(H)], axis=1)
    er_ref[...] = jnp.concatenate(
        [jnp.sum(hd[:, h * D:(h + 1) * D] * ar[h:h + 1, :], axis=1,
                 keepdims=True) for h in range(H)], axis=1)


def _scores_body(el_ref, er_ref, elt_ref, ert_ref, erk_ref, k_ref):
    el = el_ref[...]
    er = er_ref[...]
    k = jnp.max(el, axis=0, keepdims=True)
    k_ref[...] = k
    erk_ref[...] = er + k
    elt_ref[...] = el.T
    ert_ref[...] = er.T


def _table_body(hs_ref, el_ref, k_ref, t_ref):
    hs = hs_ref[...]
    a_pos = jnp.exp(el_ref[...] - k_ref[...])
    a_neg = jnp.exp(SLOPE * (el_ref[...] - k_ref[...]))
    zpad = jnp.zeros((NB, TW - D - 1), dtype=jnp.float32)
    for h in range(H):
        hs_h = hs[:, h * D:(h + 1) * D]
        ap = a_pos[:, h:h + 1]
        an = a_neg[:, h:h + 1]
        t_ref[h, 0] = jnp.concatenate([ap * hs_h, ap, zpad], axis=1)
        t_ref[h, 1] = jnp.concatenate([an * hs_h, an, zpad], axis=1)


def _epilogue_body(acc_ref, erk_ref, xd_ref, b_ref, out_ref):
    for h in range(H):
        acc_p = acc_ref[h, 0]
        acc_n = acc_ref[h, 1]
        x = erk_ref[:, h:h + 1]
        bf = jnp.exp(jnp.minimum((1.0 - SLOPE) * x, 0.0))
        bf2 = jnp.exp(jnp.minimum(-(1.0 - SLOPE) * x, 0.0))
        num = bf * acc_p[:, 0:D] + bf2 * acc_n[:, 0:D]
        den = bf * acc_p[:, D:D + 1] + bf2 * acc_n[:, D:D + 1]
        good = den > 0.0
        val = jnp.where(good, num / jnp.where(good, den, 1.0), 0.0)
        out_ref[:, h * D:(h + 1) * D] = (
            val + b_ref[:, h * D:(h + 1) * D] + xd_ref[:, h * D:(h + 1) * D])


def _edge_kernel_body(pair, t_hbm, elt_hbm, ert_hbm, src_hbm, dst_hbm, zr_hbm,
                      acc_hbm,
                      elh, erh, srcb, dstb, gidx, sidx, rows,
                      accsp, sem):
    core = lax.axis_index("c")
    tile = lax.axis_index("s")
    head = 2 * pair + core
    pltpu.sync_copy(elt_hbm.at[head], elh)
    pltpu.sync_copy(ert_hbm.at[head], erh)
    @pl.when(tile < ZTILES)
    def _zero():
        pltpu.sync_copy(zr_hbm, accsp.at[pl.ds(tile * COPY_ROWS, COPY_ROWS)])

    plsc.subcore_barrier()

    ebase = tile * EPT
    hbase = head * (2 * N)

    @pl.loop(0, NCHUNK)
    def _chunk(c):
        base = ebase + c * CHUNK
        pltpu.sync_copy(src_hbm.at[pl.ds(base, CHUNK)], srcb)
        pltpu.sync_copy(dst_hbm.at[pl.ds(base, CHUNK)], dstb)
        for j in range(CHUNK // 16):
            sv = srcb[pl.ds(j * 16, 16)]
            dv = dstb[pl.ds(j * 16, 16)]
            elv = plsc.load_gather(elh, [sv])
            erv = plsc.load_gather(erh, [dv])
            neg = (elv + erv) < 0.0
            offs = jnp.where(neg, N, 0)
            q = j * 16
            gidx[pl.ds(q, 16)] = sv + offs + hbase
            sidx[q // SLEN, pl.ds(q % SLEN, 16)] = dv + offs
        copies = []
        for s in range(NSTR):
            copies.append(pltpu.async_copy(
                t_hbm.at[gidx.at[pl.ds(s * SLEN, SLEN)]],
                rows.at[pl.ds(s * SLEN, SLEN)], sem))
        for cp in copies:
            cp.wait()
        for s in range(NSTR):
            pltpu.sync_copy(rows.at[pl.ds(s * SLEN, SLEN)],
                            accsp.at[sidx.at[s]], add=True)

    plsc.subcore_barrier()

    @pl.when(tile < ZTILES)
    def _copyout():
        pltpu.sync_copy(accsp.at[pl.ds(tile * COPY_ROWS, COPY_ROWS)],
                        acc_hbm.at[pl.ds(core * ACC_ROWS + tile * COPY_ROWS,
                                         COPY_ROWS)])


def _conv(feat_src, feat_dst, src, dst, w, attn_l, attn_r, bias):
    hs, el, er = pl.pallas_call(
        _proj_body,
        grid=(NBLK,),
        in_specs=[
            pl.BlockSpec((NB, F), lambda i: (i, 0)),
            pl.BlockSpec((NB, F), lambda i: (i, 0)),
            pl.BlockSpec((F, F), lambda i: (0, 0)),
            pl.BlockSpec((H, D), lambda i: (0, 0)),
            pl.BlockSpec((H, D), lambda i: (0, 0)),
        ],
        out_specs=[
            pl.BlockSpec((NB, F), lambda i: (i, 0)),
            pl.BlockSpec((NB, H), lambda i: (i, 0)),
            pl.BlockSpec((NB, H), lambda i: (i, 0)),
        ],
        out_shape=[
            jax.ShapeDtypeStruct((N, F), jnp.float32),
            jax.ShapeDtypeStruct((N, H), jnp.float32),
            jax.ShapeDtypeStruct((N, H), jnp.float32),
        ],
    )(feat_src, feat_dst, w, attn_l, attn_r)

    elt, ert, erk, kmax = pl.pallas_call(
        _scores_body,
        out_shape=[
            jax.ShapeDtypeStruct((H, N), jnp.float32),
            jax.ShapeDtypeStruct((H, N), jnp.float32),
            jax.ShapeDtypeStruct((N, H), jnp.float32),
            jax.ShapeDtypeStruct((1, H), jnp.float32),
        ],
    )(el, er)

    tbl = pl.pallas_call(
        _table_body,
        grid=(NBLK,),
        in_specs=[
            pl.BlockSpec((NB, F), lambda i: (i, 0)),
            pl.BlockSpec((NB, H), lambda i: (i, 0)),
            pl.BlockSpec((1, H), lambda i: (0, 0)),
        ],
        out_specs=pl.BlockSpec((H, 2, NB, TW), lambda i: (0, 0, i, 0)),
        out_shape=jax.ShapeDtypeStruct((H, 2, N, TW), jnp.float32),
    )(hs, el, kmax)

    tbl_flat = tbl.reshape(2 * H * N, TW)
    zrows = jnp.zeros((COPY_ROWS, TW), dtype=jnp.float32)

    mesh = plsc.VectorSubcoreMesh(core_axis_name="c", subcore_axis_name="s")
    cp = pltpu.CompilerParams()
    if "needs_layout_passes" in pltpu.CompilerParams.__dataclass_fields__:
        cp = dataclasses.replace(cp, needs_layout_passes=False)
    if "use_tc_tiling_on_sc" in pltpu.CompilerParams.__dataclass_fields__:
        cp = dataclasses.replace(cp, use_tc_tiling_on_sc=False)
    accs = []
    for pair in range(2):
        edge_kernel = functools.partial(
            pl.kernel,
            mesh=mesh,
            compiler_params=cp,
            out_type=jax.ShapeDtypeStruct((2 * ACC_ROWS, TW), jnp.float32),
            scratch_types=[
                pltpu.VMEM((N,), jnp.float32),
                pltpu.VMEM((N,), jnp.float32),
                pltpu.VMEM((CHUNK,), jnp.int32),
                pltpu.VMEM((CHUNK,), jnp.int32),
                pltpu.VMEM((CHUNK,), jnp.int32),
                pltpu.VMEM((NSTR, SLEN), jnp.int32),
                pltpu.VMEM((CHUNK, TW), jnp.float32),
                pltpu.VMEM_SHARED((ACC_ROWS, TW), jnp.float32),
                pltpu.SemaphoreType.DMA,
            ],
        )(functools.partial(_edge_kernel_body, pair))
        accs.append(edge_kernel(tbl_flat, elt, ert, src, dst, zrows))
    acc = jnp.concatenate(accs, axis=0)

    out = pl.pallas_call(
        _epilogue_body,
        grid=(NBLK,),
        in_specs=[
            pl.BlockSpec((H, 2, NB, TW), lambda i: (0, 0, i, 0)),
            pl.BlockSpec((NB, H), lambda i: (i, 0)),
            pl.BlockSpec((NB, F), lambda i: (i, 0)),
            pl.BlockSpec((1, F), lambda i: (0, 0)),
        ],
        out_specs=pl.BlockSpec((NB, F), lambda i: (i, 0)),
        out_shape=jax.ShapeDtypeStruct((N, H * D), jnp.float32),
    )(acc.reshape(H, 2, N, TW), erk, feat_dst, bias.reshape(1, H * D))
    return out


def kernel(x_endpoint, x_flow, edge_index_ep2flow, edge_index_flow2ep,
           W1, attn_l1, attn_r1, b1, W2, attn_l2, attn_r2, b2):
    s1 = edge_index_ep2flow[0].astype(jnp.int32)
    d1 = edge_index_ep2flow[1].astype(jnp.int32)
    s2 = edge_index_flow2ep[0].astype(jnp.int32)
    d2 = edge_index_flow2ep[1].astype(jnp.int32)
    h_flow = _conv(x_endpoint, x_flow, s1, d1, W1, attn_l1, attn_r1, b1)
    h_endpoint = _conv(x_flow, x_endpoint, s2, d2, W2, attn_l2, attn_r2, b2)
    return (h_flow, h_endpoint)


# speedup vs baseline: 68.5694x; 1.0366x over previous
import dataclasses
import functools

import jax
import jax.numpy as jnp
from jax import lax
from jax.experimental import pallas as pl
from jax.experimental.pallas import tpu as pltpu
from jax.experimental.pallas import tpu_sc as plsc

N = 10000
E = 320000
F = 128
H = 4
D = 32
TW = 32
SLOPE = 0.2

NTILE = 16
NCORE = 2
EPT = E // NTILE
CHUNK = 400
NCHUNK = EPT // CHUNK
SLEN = 80
NSTR = CHUNK // SLEN
ACC_ROWS = 2 * N
ZTILES = 10
COPY_ROWS = ACC_ROWS // ZTILES


NB = 1000
NBLK = N // NB


def _proj_body(fs_ref, fd_ref, w_ref, al_ref, ar_ref,
               hs_ref, el_ref, er_ref):
    w = w_ref[...]
    hs = jnp.dot(fs_ref[...], w, preferred_element_type=jnp.float32)
    hd = jnp.dot(fd_ref[...], w, preferred_element_type=jnp.float32)
    hs_ref[...] = hs
    al = al_ref[...]
    ar = ar_ref[...]
    el_ref[...] = jnp.concatenate(
        [jnp.sum(hs[:, h * D:(h + 1) * D] * al[h:h + 1, :], axis=1,
                 keepdims=True) for h in range(H)], axis=1)
    er_ref[...] = jnp.concatenate(
        [jnp.sum(hd[:, h * D:(h + 1) * D] * ar[h:h + 1, :], axis=1,
                 keepdims=True) for h in range(H)], axis=1)


def _scores_body(el_ref, er_ref, elt_ref, ert_ref, erk_ref, k_ref):
    el = el_ref[...]
    er = er_ref[...]
    k = jnp.max(el, axis=0, keepdims=True)
    k_ref[...] = k
    erk_ref[...] = er + k
    elt_ref[...] = (el - k).T
    ert_ref[...] = (er + k).T


def _table_body(hs_ref, el_ref, k_ref, t_ref):
    hs = hs_ref[...]
    a_pos = jnp.exp(el_ref[...] - k_ref[...])
    a_neg = jnp.exp(SLOPE * (el_ref[...] - k_ref[...]))
    for h in range(H):
        hs_h = hs[:, h * D:(h + 1) * D]
        t_ref[h, 0] = a_pos[:, h:h + 1] * hs_h
        t_ref[h, 1] = a_neg[:, h:h + 1] * hs_h


def _epilogue_body(acc_ref, den_ref, erk_ref, xd_ref, b_ref, out_ref):
    ones16 = jnp.ones((NTILE, 1), dtype=jnp.float32)
    for h in range(H):
        acc_p = acc_ref[h, 0]
        acc_n = acc_ref[h, 1]
        dp = jnp.dot(den_ref[h, 0], ones16,
                     preferred_element_type=jnp.float32)
        dn = jnp.dot(den_ref[h, 1], ones16,
                     preferred_element_type=jnp.float32)
        x = erk_ref[:, h:h + 1]
        bf = jnp.exp(jnp.minimum((1.0 - SLOPE) * x, 0.0))
        bf2 = jnp.exp(jnp.minimum(-(1.0 - SLOPE) * x, 0.0))
        num = bf * acc_p + bf2 * acc_n
        den = bf * dp + bf2 * dn
        good = den > 0.0
        val = jnp.where(good, num / jnp.where(good, den, 1.0), 0.0)
        out_ref[:, h * D:(h + 1) * D] = (
            val + b_ref[:, h * D:(h + 1) * D] + xd_ref[:, h * D:(h + 1) * D])


def _edge_kernel_body(pair, t_hbm, elt_hbm, ert_hbm, src_hbm, dst_hbm, zr_hbm,
                      zd_hbm,
                      acc_hbm, den_hbm,
                      elh, erh, srcb, dstb, gidx, sidx, rows, denom,
                      accsp, sem):
    core = lax.axis_index("c")
    tile = lax.axis_index("s")
    head = 2 * pair + core
    pltpu.sync_copy(elt_hbm.at[head], elh)
    pltpu.sync_copy(ert_hbm.at[head], erh)
    pltpu.sync_copy(zd_hbm, denom)
    @pl.when(tile < ZTILES)
    def _zero():
        pltpu.sync_copy(zr_hbm, accsp.at[pl.ds(tile * COPY_ROWS, COPY_ROWS)])

    plsc.subcore_barrier()

    ebase = tile * EPT
    hbase = head * (2 * N)

    @pl.loop(0, NCHUNK)
    def _chunk(c):
        base = ebase + c * CHUNK
        pltpu.sync_copy(src_hbm.at[pl.ds(base, CHUNK)], srcb)
        pltpu.sync_copy(dst_hbm.at[pl.ds(base, CHUNK)], dstb)
        for j in range(CHUNK // 16):
            sv = srcb[pl.ds(j * 16, 16)]
            dv = dstb[pl.ds(j * 16, 16)]
            elv = plsc.load_gather(elh, [sv])
            erv = plsc.load_gather(erh, [dv])
            neg = (elv + erv) < 0.0
            offs = jnp.where(neg, N, 0)
            a = jnp.exp(jnp.where(neg, SLOPE * elv, elv))
            plsc.addupdate_scatter(denom, [dv + offs], a)
            q = j * 16
            gidx[pl.ds(q, 16)] = sv + offs + hbase
            sidx[q // SLEN, pl.ds(q % SLEN, 16)] = dv + offs
        copies = []
        for s in range(NSTR):
            copies.append(pltpu.async_copy(
                t_hbm.at[gidx.at[pl.ds(s * SLEN, SLEN)]],
                rows.at[pl.ds(s * SLEN, SLEN)], sem))
        for cp in copies:
            cp.wait()
        for s in range(NSTR):
            pltpu.sync_copy(rows.at[pl.ds(s * SLEN, SLEN)],
                            accsp.at[sidx.at[s]], add=True)

    pltpu.sync_copy(denom, den_hbm.at[core, tile])

    plsc.subcore_barrier()

    @pl.when(tile < ZTILES)
    def _copyout():
        pltpu.sync_copy(accsp.at[pl.ds(tile * COPY_ROWS, COPY_ROWS)],
                        acc_hbm.at[pl.ds(core * ACC_ROWS + tile * COPY_ROWS,
                                         COPY_ROWS)])


def _conv(feat_src, feat_dst, src, dst, w, attn_l, attn_r, bias):
    hs, el, er = pl.pallas_call(
        _proj_body,
        grid=(NBLK,),
        in_specs=[
            pl.BlockSpec((NB, F), lambda i: (i, 0)),
            pl.BlockSpec((NB, F), lambda i: (i, 0)),
            pl.BlockSpec((F, F), lambda i: (0, 0)),
            pl.BlockSpec((H, D), lambda i: (0, 0)),
            pl.BlockSpec((H, D), lambda i: (0, 0)),
        ],
        out_specs=[
            pl.BlockSpec((NB, F), lambda i: (i, 0)),
            pl.BlockSpec((NB, H), lambda i: (i, 0)),
            pl.BlockSpec((NB, H), lambda i: (i, 0)),
        ],
        out_shape=[
            jax.ShapeDtypeStruct((N, F), jnp.float32),
            jax.ShapeDtypeStruct((N, H), jnp.float32),
            jax.ShapeDtypeStruct((N, H), jnp.float32),
        ],
    )(feat_src, feat_dst, w, attn_l, attn_r)

    elt, ert, erk, kmax = pl.pallas_call(
        _scores_body,
        out_shape=[
            jax.ShapeDtypeStruct((H, N), jnp.float32),
            jax.ShapeDtypeStruct((H, N), jnp.float32),
            jax.ShapeDtypeStruct((N, H), jnp.float32),
            jax.ShapeDtypeStruct((1, H), jnp.float32),
        ],
    )(el, er)

    tbl = pl.pallas_call(
        _table_body,
        grid=(NBLK,),
        in_specs=[
            pl.BlockSpec((NB, F), lambda i: (i, 0)),
            pl.BlockSpec((NB, H), lambda i: (i, 0)),
            pl.BlockSpec((1, H), lambda i: (0, 0)),
        ],
        out_specs=pl.BlockSpec((H, 2, NB, TW), lambda i: (0, 0, i, 0)),
        out_shape=jax.ShapeDtypeStruct((H, 2, N, TW), jnp.float32),
    )(hs, el, kmax)

    tbl_flat = tbl.reshape(2 * H * N, TW)
    zrows = jnp.zeros((COPY_ROWS, TW), dtype=jnp.float32)
    zden = jnp.zeros((2 * N,), dtype=jnp.float32)

    mesh = plsc.VectorSubcoreMesh(core_axis_name="c", subcore_axis_name="s")
    cp = pltpu.CompilerParams()
    if "needs_layout_passes" in pltpu.CompilerParams.__dataclass_fields__:
        cp = dataclasses.replace(cp, needs_layout_passes=False)
    if "use_tc_tiling_on_sc" in pltpu.CompilerParams.__dataclass_fields__:
        cp = dataclasses.replace(cp, use_tc_tiling_on_sc=False)
    accs = []
    dens = []
    for pair in range(2):
        edge_kernel = functools.partial(
            pl.kernel,
            mesh=mesh,
            compiler_params=cp,
            out_type=[
                jax.ShapeDtypeStruct((2 * ACC_ROWS, TW), jnp.float32),
                jax.ShapeDtypeStruct((NCORE, NTILE, 2 * N), jnp.float32),
            ],
            scratch_types=[
                pltpu.VMEM((N,), jnp.float32),
                pltpu.VMEM((N,), jnp.float32),
                pltpu.VMEM((CHUNK,), jnp.int32),
                pltpu.VMEM((CHUNK,), jnp.int32),
                pltpu.VMEM((CHUNK,), jnp.int32),
                pltpu.VMEM((NSTR, SLEN), jnp.int32),
                pltpu.VMEM((CHUNK, TW), jnp.float32),
                pltpu.VMEM((2 * N,), jnp.float32),
                pltpu.VMEM_SHARED((ACC_ROWS, TW), jnp.float32),
                pltpu.SemaphoreType.DMA,
            ],
        )(functools.partial(_edge_kernel_body, pair))
        a, dn = edge_kernel(tbl_flat, elt, ert, src, dst, zrows, zden)
        accs.append(a)
        dens.append(dn)
    acc = jnp.concatenate(accs, axis=0)
    den = jnp.transpose(jnp.stack(dens).reshape(H, NTILE, 2, N), (0, 2, 3, 1))

    out = pl.pallas_call(
        _epilogue_body,
        grid=(NBLK,),
        in_specs=[
            pl.BlockSpec((H, 2, NB, TW), lambda i: (0, 0, i, 0)),
            pl.BlockSpec((H, 2, NB, NTILE), lambda i: (0, 0, i, 0)),
            pl.BlockSpec((NB, H), lambda i: (i, 0)),
            pl.BlockSpec((NB, F), lambda i: (i, 0)),
            pl.BlockSpec((1, F), lambda i: (0, 0)),
        ],
        out_specs=pl.BlockSpec((NB, F), lambda i: (i, 0)),
        out_shape=jax.ShapeDtypeStruct((N, H * D), jnp.float32),
    )(acc.reshape(H, 2, N, TW), den, erk, feat_dst, bias.reshape(1, H * D))
    return out


def kernel(x_endpoint, x_flow, edge_index_ep2flow, edge_index_flow2ep,
           W1, attn_l1, attn_r1, b1, W2, attn_l2, attn_r2, b2):
    s1 = edge_index_ep2flow[0].astype(jnp.int32)
    d1 = edge_index_ep2flow[1].astype(jnp.int32)
    s2 = edge_index_flow2ep[0].astype(jnp.int32)
    d2 = edge_index_flow2ep[1].astype(jnp.int32)
    h_flow = _conv(x_endpoint, x_flow, s1, d1, W1, attn_l1, attn_r1, b1)
    h_endpoint = _conv(x_flow, x_endpoint, s2, d2, W2, attn_l2, attn_r2, b2)
    return (h_flow, h_endpoint)


# speedup vs baseline: 84.4275x; 1.2313x over previous
import dataclasses
import functools

import jax
import jax.numpy as jnp
from jax import lax
from jax.experimental import pallas as pl
from jax.experimental.pallas import tpu as pltpu
from jax.experimental.pallas import tpu_sc as plsc

N = 10000
E = 320000
F = 128
H = 4
D = 32
TW = 32
SLOPE = 0.2

NTILE = 16
NCORE = 2
EPT = E // NTILE
CHUNK = 400
NCHUNK = EPT // CHUNK
SLEN = 80
NSTR = CHUNK // SLEN
ACC_ROWS = 2 * N
ZTILES = 10
COPY_ROWS = ACC_ROWS // ZTILES


NB = 1000
NBLK = N // NB


def _proj_body(fs_ref, fd_ref, w_ref, al_ref, ar_ref,
               hs_ref, el_ref, er_ref):
    w = w_ref[...]
    hs = jnp.dot(fs_ref[...], w, preferred_element_type=jnp.float32)
    hd = jnp.dot(fd_ref[...], w, preferred_element_type=jnp.float32)
    hs_ref[...] = hs
    al = al_ref[...]
    ar = ar_ref[...]
    el_ref[...] = jnp.concatenate(
        [jnp.sum(hs[:, h * D:(h + 1) * D] * al[h:h + 1, :], axis=1,
                 keepdims=True) for h in range(H)], axis=1)
    er_ref[...] = jnp.concatenate(
        [jnp.sum(hd[:, h * D:(h + 1) * D] * ar[h:h + 1, :], axis=1,
                 keepdims=True) for h in range(H)], axis=1)


def _scores_body(el_ref, er_ref, elt_ref, ert_ref, erk_ref, k_ref):
    el = el_ref[...]
    er = er_ref[...]
    k = jnp.max(el, axis=0, keepdims=True)
    k_ref[...] = k
    erk_ref[...] = er + k
    elt_ref[...] = (el - k).T
    ert_ref[...] = (er + k).T


def _table_body(hs_ref, el_ref, k_ref, t_ref):
    hs = hs_ref[...]
    a_pos = jnp.exp(el_ref[...] - k_ref[...])
    a_neg = jnp.exp(SLOPE * (el_ref[...] - k_ref[...]))
    for h in range(H):
        hs_h = hs[:, h * D:(h + 1) * D]
        t_ref[h, 0] = a_pos[:, h:h + 1] * hs_h
        t_ref[h, 1] = a_neg[:, h:h + 1] * hs_h


def _epilogue_body(acc_ref, den_ref, erk_ref, xd_ref, b_ref, out_ref):
    ones16 = jnp.ones((NTILE, 1), dtype=jnp.float32)
    for h in range(H):
        acc_p = acc_ref[h, 0]
        acc_n = acc_ref[h, 1]
        dp = jnp.dot(den_ref[h, 0], ones16,
                     preferred_element_type=jnp.float32)
        dn = jnp.dot(den_ref[h, 1], ones16,
                     preferred_element_type=jnp.float32)
        x = erk_ref[:, h:h + 1]
        bf = jnp.exp(jnp.minimum((1.0 - SLOPE) * x, 0.0))
        bf2 = jnp.exp(jnp.minimum(-(1.0 - SLOPE) * x, 0.0))
        num = bf * acc_p + bf2 * acc_n
        den = bf * dp + bf2 * dn
        good = den > 0.0
        val = jnp.where(good, num / jnp.where(good, den, 1.0), 0.0)
        out_ref[:, h * D:(h + 1) * D] = (
            val + b_ref[:, h * D:(h + 1) * D] + xd_ref[:, h * D:(h + 1) * D])


def _edge_kernel_body(pair, t_hbm, elt_hbm, ert_hbm, src_hbm, dst_hbm, zr_hbm,
                      zd_hbm,
                      acc_hbm, den_hbm,
                      elh, erh, srcb, dstb, gidx, sidx, rows, denom,
                      accsp, sem0, sem1):
    sems = (sem0, sem1)
    core = lax.axis_index("c")
    tile = lax.axis_index("s")
    head = 2 * pair + core
    pltpu.sync_copy(elt_hbm.at[head], elh)
    pltpu.sync_copy(ert_hbm.at[head], erh)
    pltpu.sync_copy(zd_hbm, denom)
    @pl.when(tile < ZTILES)
    def _zero():
        pltpu.sync_copy(zr_hbm, accsp.at[pl.ds(tile * COPY_ROWS, COPY_ROWS)])

    plsc.subcore_barrier()

    ebase = tile * EPT
    hbase = head * (2 * N)

    def _fire(c, b):
        base = ebase + c * CHUNK
        pltpu.sync_copy(src_hbm.at[pl.ds(base, CHUNK)], srcb)
        pltpu.sync_copy(dst_hbm.at[pl.ds(base, CHUNK)], dstb)
        for j in range(CHUNK // 16):
            sv = srcb[pl.ds(j * 16, 16)]
            dv = dstb[pl.ds(j * 16, 16)]
            elv = plsc.load_gather(elh, [sv])
            erv = plsc.load_gather(erh, [dv])
            neg = (elv + erv) < 0.0
            offs = jnp.where(neg, N, 0)
            a = jnp.exp(jnp.where(neg, SLOPE * elv, elv))
            plsc.addupdate_scatter(denom, [dv + offs], a)
            q = j * 16
            gidx[b, pl.ds(q, 16)] = sv + offs + hbase
            sidx[b, q // SLEN, pl.ds(q % SLEN, 16)] = dv + offs
        for s in range(NSTR):
            pltpu.async_copy(
                t_hbm.at[gidx.at[b, pl.ds(s * SLEN, SLEN)]],
                rows.at[b, pl.ds(s * SLEN, SLEN)], sems[b])

    def _drain_scatter(b):
        pltpu.make_async_copy(t_hbm.at[pl.ds(0, CHUNK)],
                              rows.at[b], sems[b]).wait()
        for s in range(NSTR):
            pltpu.sync_copy(rows.at[b, pl.ds(s * SLEN, SLEN)],
                            accsp.at[sidx.at[b, s]], add=True)

    _fire(0, 0)

    @pl.loop(0, NCHUNK, step=2)
    def _chunk(c):
        for b in range(2):
            cc = c + b

            @pl.when(cc + 1 < NCHUNK)
            def _next():
                _fire(cc + 1, 1 - b)

            _drain_scatter(b)

    pltpu.sync_copy(denom, den_hbm.at[core, tile])

    plsc.subcore_barrier()

    @pl.when(tile < ZTILES)
    def _copyout():
        pltpu.sync_copy(accsp.at[pl.ds(tile * COPY_ROWS, COPY_ROWS)],
                        acc_hbm.at[pl.ds(core * ACC_ROWS + tile * COPY_ROWS,
                                         COPY_ROWS)])


def _conv(feat_src, feat_dst, src, dst, w, attn_l, attn_r, bias):
    hs, el, er = pl.pallas_call(
        _proj_body,
        grid=(NBLK,),
        in_specs=[
            pl.BlockSpec((NB, F), lambda i: (i, 0)),
            pl.BlockSpec((NB, F), lambda i: (i, 0)),
            pl.BlockSpec((F, F), lambda i: (0, 0)),
            pl.BlockSpec((H, D), lambda i: (0, 0)),
            pl.BlockSpec((H, D), lambda i: (0, 0)),
        ],
        out_specs=[
            pl.BlockSpec((NB, F), lambda i: (i, 0)),
            pl.BlockSpec((NB, H), lambda i: (i, 0)),
            pl.BlockSpec((NB, H), lambda i: (i, 0)),
        ],
        out_shape=[
            jax.ShapeDtypeStruct((N, F), jnp.float32),
            jax.ShapeDtypeStruct((N, H), jnp.float32),
            jax.ShapeDtypeStruct((N, H), jnp.float32),
        ],
    )(feat_src, feat_dst, w, attn_l, attn_r)

    elt, ert, erk, kmax = pl.pallas_call(
        _scores_body,
        out_shape=[
            jax.ShapeDtypeStruct((H, N), jnp.float32),
            jax.ShapeDtypeStruct((H, N), jnp.float32),
            jax.ShapeDtypeStruct((N, H), jnp.float32),
            jax.ShapeDtypeStruct((1, H), jnp.float32),
        ],
    )(el, er)

    tbl = pl.pallas_call(
        _table_body,
        grid=(NBLK,),
        in_specs=[
            pl.BlockSpec((NB, F), lambda i: (i, 0)),
            pl.BlockSpec((NB, H), lambda i: (i, 0)),
            pl.BlockSpec((1, H), lambda i: (0, 0)),
        ],
        out_specs=pl.BlockSpec((H, 2, NB, TW), lambda i: (0, 0, i, 0)),
        out_shape=jax.ShapeDtypeStruct((H, 2, N, TW), jnp.float32),
    )(hs, el, kmax)

    tbl_flat = tbl.reshape(2 * H * N, TW)
    zrows = jnp.zeros((COPY_ROWS, TW), dtype=jnp.float32)
    zden = jnp.zeros((2 * N,), dtype=jnp.float32)

    mesh = plsc.VectorSubcoreMesh(core_axis_name="c", subcore_axis_name="s")
    cp = pltpu.CompilerParams()
    if "needs_layout_passes" in pltpu.CompilerParams.__dataclass_fields__:
        cp = dataclasses.replace(cp, needs_layout_passes=False)
    if "use_tc_tiling_on_sc" in pltpu.CompilerParams.__dataclass_fields__:
        cp = dataclasses.replace(cp, use_tc_tiling_on_sc=False)
    accs = []
    dens = []
    for pair in range(2):
        edge_kernel = functools.partial(
            pl.kernel,
            mesh=mesh,
            compiler_params=cp,
            out_type=[
                jax.ShapeDtypeStruct((2 * ACC_ROWS, TW), jnp.float32),
                jax.ShapeDtypeStruct((NCORE, NTILE, 2 * N), jnp.float32),
            ],
            scratch_types=[
                pltpu.VMEM((N,), jnp.float32),
                pltpu.VMEM((N,), jnp.float32),
                pltpu.VMEM((CHUNK,), jnp.int32),
                pltpu.VMEM((CHUNK,), jnp.int32),
                pltpu.VMEM((2, CHUNK), jnp.int32),
                pltpu.VMEM((2, NSTR, SLEN), jnp.int32),
                pltpu.VMEM((2, CHUNK, TW), jnp.float32),
                pltpu.VMEM((2 * N,), jnp.float32),
                pltpu.VMEM_SHARED((ACC_ROWS, TW), jnp.float32),
                pltpu.SemaphoreType.DMA,
                pltpu.SemaphoreType.DMA,
            ],
        )(functools.partial(_edge_kernel_body, pair))
        a, dn = edge_kernel(tbl_flat, elt, ert, src, dst, zrows, zden)
        accs.append(a)
        dens.append(dn)
    acc = jnp.concatenate(accs, axis=0)
    den = jnp.transpose(jnp.stack(dens).reshape(H, NTILE, 2, N), (0, 2, 3, 1))

    out = pl.pallas_call(
        _epilogue_body,
        grid=(NBLK,),
        in_specs=[
            pl.BlockSpec((H, 2, NB, TW), lambda i: (0, 0, i, 0)),
            pl.BlockSpec((H, 2, NB, NTILE), lambda i: (0, 0, i, 0)),
            pl.BlockSpec((NB, H), lambda i: (i, 0)),
            pl.BlockSpec((NB, F), lambda i: (i, 0)),
            pl.BlockSpec((1, F), lambda i: (0, 0)),
        ],
        out_specs=pl.BlockSpec((NB, F), lambda i: (i, 0)),
        out_shape=jax.ShapeDtypeStruct((N, H * D), jnp.float32),
    )(acc.reshape(H, 2, N, TW), den, erk, feat_dst, bias.reshape(1, H * D))
    return out


def kernel(x_endpoint, x_flow, edge_index_ep2flow, edge_index_flow2ep,
           W1, attn_l1, attn_r1, b1, W2, attn_l2, attn_r2, b2):
    s1 = edge_index_ep2flow[0].astype(jnp.int32)
    d1 = edge_index_ep2flow[1].astype(jnp.int32)
    s2 = edge_index_flow2ep[0].astype(jnp.int32)
    d2 = edge_index_flow2ep[1].astype(jnp.int32)
    h_flow = _conv(x_endpoint, x_flow, s1, d1, W1, attn_l1, attn_r1, b1)
    h_endpoint = _conv(x_flow, x_endpoint, s2, d2, W2, attn_l2, attn_r2, b2)
    return (h_flow, h_endpoint)
